# baseline (speedup 1.0000x reference)
import jax
import jax.numpy as jnp
from jax import lax
from jax.experimental import pallas as pl
from jax.experimental.pallas import tpu as pltpu
from jax.experimental.pallas import tpu_sc as plsc

D = 64
BATCH = 4096
SEQ = 200
CHUNK = 400
NB = 4

_info = plsc.get_sparse_core_info()
NC, NS, L = _info.num_cores, _info.num_subcores, _info.num_lanes
NW = NC * NS
ROWS_PER_W = BATCH * SEQ // NW
CHUNKS_PER_W = ROWS_PER_W // CHUNK


def _body(x_hbm, table_hbm, posaux_hbm, out_hbm,
          idx0, idx1, idx2, idx3, qb0, qb1, qb2, qb3,
          rows0, rows1, rows2, rows3,
          isem0, isem1, isem2, isem3,
          qsem0, qsem1, qsem2, qsem3,
          gsem0, gsem1, gsem2, gsem3,
          ssem0, ssem1, ssem2, ssem3):
    wid = lax.axis_index("s") * NC + lax.axis_index("c")
    wbase = wid * ROWS_PER_W

    idxb = (idx0, idx1, idx2, idx3)
    qb = (qb0, qb1, qb2, qb3)
    rows = (rows0, rows1, rows2, rows3)
    isem = (isem0, isem1, isem2, isem3)
    qsem = (qsem0, qsem1, qsem2, qsem3)
    gsem = (gsem0, gsem1, gsem2, gsem3)
    ssem = (ssem0, ssem1, ssem2, ssem3)

    def issue_idxcopy(c, b):
        pltpu.async_copy(x_hbm.at[pl.ds(wbase + c * CHUNK, CHUNK)], idxb[b],
                         isem[b])

    def wait_idxcopy(b):
        pltpu.make_async_copy(x_hbm.at[pl.ds(0, CHUNK)], idxb[b],
                              isem[b]).wait()

    def compute_q(b):
        def grp(g2, carry):
            ivec = idxb[b][pl.ds(g2 * L, L)]
            svec = lax.iota(jnp.int32, L) + g2 * L
            svec = jnp.where(svec >= SEQ, svec - SEQ, svec)
            qb[b][pl.ds(g2 * L, L)] = jnp.where(ivec == 2, SEQ + svec, svec)
            return carry

        lax.fori_loop(0, CHUNK // L, grp, 0)

    def issue_qgather(b):
        pltpu.async_copy(posaux_hbm.at[qb[b]], rows[b], qsem[b])

    def wait_qgather(b):
        pltpu.make_async_copy(out_hbm.at[pl.ds(0, CHUNK), :], rows[b],
                              qsem[b]).wait()

    def issue_gather(b):
        pltpu.async_copy(table_hbm.at[idxb[b]], rows[b], gsem[b], add=True)

    def wait_gather(b):
        pltpu.make_async_copy(out_hbm.at[pl.ds(0, CHUNK), :], rows[b],
                              gsem[b]).wait()

    def issue_scatter(c, b):
        pltpu.async_copy(rows[b],
                         out_hbm.at[pl.ds(wbase + c * CHUNK, CHUNK), :],
                         ssem[b])

    def wait_scatter(b):
        pltpu.make_async_copy(rows[b], out_hbm.at[pl.ds(0, CHUNK), :],
                              ssem[b]).wait()

    N = CHUNKS_PER_W
    issue_idxcopy(0, 0)
    issue_idxcopy(1, 1)
    issue_idxcopy(2, 2)
    wait_idxcopy(0)
    compute_q(0)
    issue_qgather(0)
    wait_idxcopy(1)
    compute_q(1)
    issue_qgather(1)
    wait_qgather(0)
    issue_gather(0)

    def quad_body(gi, carry):
        for b in range(NB):
            c = gi * NB + b
            s1 = (b + 1) % NB
            s2 = (b + 2) % NB
            s3 = (b + 3) % NB

            @pl.when(c + 3 < N)
            def _():
                issue_idxcopy(c + 3, s3)

            @pl.when(c + 2 < N)
            def _():
                wait_idxcopy(s2)
                compute_q(s2)

                @pl.when(c >= 2)
                def _():
                    wait_scatter(s2)
                issue_qgather(s2)

            @pl.when(c + 1 < N)
            def _():
                wait_qgather(s1)
                issue_gather(s1)

            wait_gather(b)
            issue_scatter(c, b)
        return carry

    lax.fori_loop(0, N // NB, quad_body, 0)
    for b in range(NB):
        wait_scatter(b)


@jax.jit
def _run(xf, table, posaux):
    fn = pl.kernel(
        _body,
        mesh=plsc.VectorSubcoreMesh(core_axis_name="c", subcore_axis_name="s"),
        compiler_params=pltpu.CompilerParams(use_tc_tiling_on_sc=False),
        out_type=jax.ShapeDtypeStruct((BATCH * SEQ, D), jnp.float32),
        scratch_types=(
            [pltpu.VMEM((CHUNK,), jnp.int32)] * 8
            + [pltpu.VMEM((CHUNK, D), jnp.float32)] * 4
            + [pltpu.SemaphoreType.DMA] * 16
        ),
    )
    return fn(xf, table, posaux)


def kernel(x, table, pos_enc):
    xf = x.reshape(BATCH * SEQ)
    posaux = jnp.concatenate([pos_enc, pos_enc - table[2]], axis=0)
    out = _run(xf, table, posaux)
    return out.reshape(BATCH, SEQ, D)
